# baseline (speedup 1.0000x reference)
import functools

import jax
import jax.numpy as jnp
from jax import lax
from jax.experimental import pallas as pl
from jax.experimental.pallas import tpu as pltpu
from jax.experimental.pallas import tpu_sc as plsc

N_NODES = 10000
N_EDGES = 320000
IN_DIM = 128
HID = 64
PAY = 128

NC, NS = 2, 16
NW = NC * NS
EPT = N_EDGES // NW
K = 72
NBLK = 144
CH = 8
NWIN = NBLK // CH
WSLOT = 3
N_PAD = 10112
ROWS_PAD = N_PAD // NS
DEG_PAD = 12288
DEG_ROWS = DEG_PAD // NS

_mesh = functools.partial(
    plsc.VectorSubcoreMesh, core_axis_name="c", subcore_axis_name="s",
    num_cores=NC, num_subcores=NS)


def _deg_body(dst_hbm, zeros_hbm, out_hbm, dst_v, ones_v, deg_sh, sem):
    cid = lax.axis_index("c")
    sid = lax.axis_index("s")
    wid = cid * NS + sid
    pltpu.sync_copy(dst_hbm.at[wid], dst_v)
    for i in range(8):
        ones_v[pl.ds(i * 16, 16)] = jnp.ones((16,), jnp.float32)
    pltpu.sync_copy(zeros_hbm.at[pl.ds(sid * DEG_ROWS, DEG_ROWS)],
                    deg_sh.at[pl.ds(sid * DEG_ROWS, DEG_ROWS)])
    plsc.subcore_barrier()

    def body(j, carry):
        pltpu.async_copy(ones_v.at[pl.ds(0, K)], deg_sh.at[dst_v.at[j]], sem,
                         add=True)
        return carry

    lax.fori_loop(0, NBLK, body, 0)

    def drain(j, carry):
        pltpu.make_async_copy(ones_v.at[pl.ds(0, K)], deg_sh.at[dst_v.at[0]],
                              sem).wait()
        return carry

    lax.fori_loop(0, NBLK, drain, 0)
    plsc.subcore_barrier()
    pltpu.sync_copy(deg_sh.at[pl.ds(sid * DEG_ROWS, DEG_ROWS)],
                    out_hbm.at[pl.ds(cid * DEG_PAD + sid * DEG_ROWS, DEG_ROWS)])


_deg_call = functools.partial(
    pl.kernel,
    _deg_body,
    out_type=jax.ShapeDtypeStruct((NC * DEG_PAD,), jnp.float32),
    scratch_types=[
        pltpu.VMEM((NBLK, K), jnp.int32),
        pltpu.VMEM((128,), jnp.float32),
        pltpu.VMEM_SHARED((DEG_PAD,), jnp.float32),
        pltpu.SemaphoreType.DMA,
    ],
)


def _agg_body(hs_hbm, src_hbm, dst_hbm, zeros_hbm, out_hbm,
              srcw, dstw, buf0, buf1, buf2, buf3, agg_sh,
              semg0, semg1, semg2, semg3, sems0, sems1, sems2, sems3,
              semw0, semw1):
    cid = lax.axis_index("c")
    sid = lax.axis_index("s")
    wid = cid * NS + sid
    pltpu.sync_copy(zeros_hbm.at[pl.ds(sid * ROWS_PAD, ROWS_PAD)],
                    agg_sh.at[pl.ds(sid * ROWS_PAD, ROWS_PAD)])
    pltpu.sync_copy(src_hbm.at[wid, pl.ds(0, CH)], srcw.at[pl.ds(0, CH)])
    pltpu.sync_copy(dst_hbm.at[wid, pl.ds(0, CH)], dstw.at[pl.ds(0, CH)])
    pltpu.sync_copy(src_hbm.at[wid, pl.ds(CH, CH)], srcw.at[pl.ds(CH, CH)])
    pltpu.sync_copy(dst_hbm.at[wid, pl.ds(CH, CH)], dstw.at[pl.ds(CH, CH)])
    plsc.subcore_barrier()

    def src_row(j):
        return srcw.at[lax.rem(j // CH, WSLOT) * CH + lax.rem(j, CH)]

    def dst_row(j):
        return dstw.at[lax.rem(j // CH, WSLOT) * CH + lax.rem(j, CH)]

    def wait_gather(buf, sem):
        pltpu.make_async_copy(hs_hbm.at[src_row(0)], buf, sem).wait()

    def wait_scatter(buf, sem):
        pltpu.make_async_copy(buf, agg_sh.at[dst_row(0)], sem).wait()

    def fire_window(w, sem):
        slot = lax.rem(w, WSLOT) * CH
        pltpu.async_copy(src_hbm.at[wid, pl.ds(w * CH, CH)],
                         srcw.at[pl.ds(slot, CH)], sem)
        pltpu.async_copy(dst_hbm.at[wid, pl.ds(w * CH, CH)],
                         dstw.at[pl.ds(slot, CH)], sem)

    def wait_window(sem):
        pltpu.make_async_copy(src_hbm.at[wid, pl.ds(0, CH)],
                              srcw.at[pl.ds(0, CH)], sem).wait()
        pltpu.make_async_copy(dst_hbm.at[wid, pl.ds(0, CH)],
                              dstw.at[pl.ds(0, CH)], sem).wait()

    pltpu.async_copy(hs_hbm.at[src_row(0)], buf0, semg0)
    pltpu.async_copy(hs_hbm.at[src_row(1)], buf1, semg1)

    bufs = (buf0, buf1, buf2, buf3)
    semg = (semg0, semg1, semg2, semg3)
    sems = (sems0, sems1, sems2, sems3)

    def body(j, carry):
        g = j + 2

        @pl.when(lax.rem(g, CH) == 0)
        def _():
            w = g // CH

            @pl.when((w >= 2) & (w < NWIN) & (lax.rem(w, 2) == 0))
            def _():
                wait_window(semw0)

            @pl.when((w >= 2) & (w < NWIN) & (lax.rem(w, 2) == 1))
            def _():
                wait_window(semw1)

            @pl.when((w + 1 < NWIN) & (lax.rem(w + 1, 2) == 0))
            def _():
                fire_window(w + 1, semw0)

            @pl.when((w + 1 < NWIN) & (lax.rem(w + 1, 2) == 1))
            def _():
                fire_window(w + 1, semw1)

        for x in range(4):
            wb = (x + 2) % 4

            @pl.when(lax.rem(j, 4) == x)
            def _(x=x, wb=wb):
                wait_gather(bufs[x], semg[x])
                pltpu.async_copy(bufs[x], agg_sh.at[dst_row(j)], sems[x],
                                 add=True)

                @pl.when(j >= 2)
                def _():
                    wait_scatter(bufs[wb], sems[wb])

                @pl.when(g < NBLK)
                def _():
                    pltpu.async_copy(hs_hbm.at[src_row(g)], bufs[wb],
                                     semg[wb])

        return carry

    lax.fori_loop(0, NBLK, body, 0)
    wait_scatter(buf2, sems2)
    wait_scatter(buf3, sems3)
    plsc.subcore_barrier()
    pltpu.sync_copy(agg_sh.at[pl.ds(sid * ROWS_PAD, ROWS_PAD)],
                    out_hbm.at[cid, pl.ds(sid * ROWS_PAD, ROWS_PAD)])


_agg_call = functools.partial(
    pl.kernel,
    _agg_body,
    out_type=jax.ShapeDtypeStruct((NC, N_PAD, PAY), jnp.float32),
    scratch_types=[
        pltpu.VMEM((WSLOT * CH, K), jnp.int32),
        pltpu.VMEM((WSLOT * CH, K), jnp.int32),
        pltpu.VMEM((K, PAY), jnp.float32),
        pltpu.VMEM((K, PAY), jnp.float32),
        pltpu.VMEM((K, PAY), jnp.float32),
        pltpu.VMEM((K, PAY), jnp.float32),
        pltpu.VMEM_SHARED((N_PAD, PAY), jnp.float32),
        pltpu.SemaphoreType.DMA,
        pltpu.SemaphoreType.DMA,
        pltpu.SemaphoreType.DMA,
        pltpu.SemaphoreType.DMA,
        pltpu.SemaphoreType.DMA,
        pltpu.SemaphoreType.DMA,
        pltpu.SemaphoreType.DMA,
        pltpu.SemaphoreType.DMA,
        pltpu.SemaphoreType.DMA,
        pltpu.SemaphoreType.DMA,
    ],
)


_RB = 632
_GRID = N_PAD // _RB


def _mm1_body(x_ref, w1_ref, h1_ref):
    h1_ref[...] = jnp.dot(x_ref[...], w1_ref[...],
                          preferred_element_type=jnp.float32)


def _mm1_call(x, W1):
    return pl.pallas_call(
        _mm1_body,
        grid=(_GRID,),
        in_specs=[
            pl.BlockSpec((_RB, IN_DIM), lambda i: (i, 0)),
            pl.BlockSpec((IN_DIM, HID), lambda i: (0, 0)),
        ],
        out_specs=pl.BlockSpec((_RB, HID), lambda i: (i, 0)),
        out_shape=jax.ShapeDtypeStruct((N_PAD, HID), jnp.float32),
    )(x, W1)


def _prep_body(degp_ref, h1_ref, hs1_ref, dis_ref, inv_ref):
    deg = degp_ref[0, 0, :] + degp_ref[0, 1, :] + 1.0
    inv = 1.0 / deg
    dis = lax.rsqrt(deg)
    h1 = h1_ref[...]
    hs1 = h1 * dis[:, None]
    hs1_ref[...] = jnp.pad(hs1, ((0, 0), (0, PAY - HID)))
    dis_ref[...] = dis[:, None]
    inv_ref[...] = inv[:, None]


def _prep_call(deg_p, h1):
    return pl.pallas_call(
        _prep_body,
        grid=(_GRID,),
        in_specs=[
            pl.BlockSpec((1, NC, _RB), lambda i: (i, 0, 0)),
            pl.BlockSpec((_RB, HID), lambda i: (i, 0)),
        ],
        out_specs=[
            pl.BlockSpec((_RB, PAY), lambda i: (i, 0)),
            pl.BlockSpec((_RB, 1), lambda i: (i, 0)),
            pl.BlockSpec((_RB, 1), lambda i: (i, 0)),
        ],
        out_shape=[
            jax.ShapeDtypeStruct((N_PAD, PAY), jnp.float32),
            jax.ShapeDtypeStruct((N_PAD, 1), jnp.float32),
            jax.ShapeDtypeStruct((N_PAD, 1), jnp.float32),
        ],
    )(deg_p, h1)


def _mid_body(aggp_ref, h1_ref, dis_ref, inv_ref, b1_ref, w2_ref,
              h2_ref, hs2_ref):
    agg = aggp_ref[0, :, :HID] + aggp_ref[1, :, :HID]
    z1 = jnp.maximum(
        dis_ref[...] * agg + h1_ref[...] * inv_ref[...] + b1_ref[...], 0.0)
    h2 = jnp.dot(z1, w2_ref[...], preferred_element_type=jnp.float32)
    h2_ref[...] = h2
    hs2_ref[...] = jnp.pad(h2 * dis_ref[...], ((0, 0), (0, PAY - HID)))


def _mid_call(agg_p, h1, dis, inv, b1, W2):
    return pl.pallas_call(
        _mid_body,
        grid=(_GRID,),
        in_specs=[
            pl.BlockSpec((NC, _RB, PAY), lambda i: (0, i, 0)),
            pl.BlockSpec((_RB, HID), lambda i: (i, 0)),
            pl.BlockSpec((_RB, 1), lambda i: (i, 0)),
            pl.BlockSpec((_RB, 1), lambda i: (i, 0)),
            pl.BlockSpec((1, HID), lambda i: (0, 0)),
            pl.BlockSpec((HID, HID), lambda i: (0, 0)),
        ],
        out_specs=[
            pl.BlockSpec((_RB, HID), lambda i: (i, 0)),
            pl.BlockSpec((_RB, PAY), lambda i: (i, 0)),
        ],
        out_shape=[
            jax.ShapeDtypeStruct((N_PAD, HID), jnp.float32),
            jax.ShapeDtypeStruct((N_PAD, PAY), jnp.float32),
        ],
    )(agg_p, h1, dis, inv, b1, W2)


def _fin_body(aggp_ref, h2_ref, dis_ref, inv_ref, b2_ref, out_ref):
    agg = aggp_ref[0, :, :HID] + aggp_ref[1, :, :HID]
    out_ref[...] = (dis_ref[...] * agg + h2_ref[...] * inv_ref[...]
                    + b2_ref[...])


def _fin_call(agg_p, h2, dis, inv, b2):
    return pl.pallas_call(
        _fin_body,
        grid=(_GRID,),
        in_specs=[
            pl.BlockSpec((NC, _RB, PAY), lambda i: (0, i, 0)),
            pl.BlockSpec((_RB, HID), lambda i: (i, 0)),
            pl.BlockSpec((_RB, 1), lambda i: (i, 0)),
            pl.BlockSpec((_RB, 1), lambda i: (i, 0)),
            pl.BlockSpec((1, HID), lambda i: (0, 0)),
        ],
        out_specs=pl.BlockSpec((_RB, HID), lambda i: (i, 0)),
        out_shape=jax.ShapeDtypeStruct((N_PAD, HID), jnp.float32),
    )(agg_p, h2, dis, inv, b2)


def kernel(x, edge_index, W1, b1, W2, b2):
    ei = edge_index.astype(jnp.int32)
    n_dummy = NBLK * K - EPT
    src = jnp.concatenate(
        [ei[0].reshape(NW, EPT),
         jnp.zeros((NW, n_dummy), jnp.int32)], axis=1).reshape(NW, NBLK, K)
    junk = (jnp.arange(n_dummy, dtype=jnp.int32) % (N_PAD - N_NODES)
            ) + N_NODES
    dst = jnp.concatenate(
        [ei[1].reshape(NW, EPT),
         jnp.tile(junk[None, :], (NW, 1))], axis=1).reshape(NW, NBLK, K)
    x_pad = jnp.pad(x, ((0, N_PAD - N_NODES), (0, 0)))
    zeros1 = jnp.zeros((DEG_PAD,), jnp.float32)
    zeros2 = jnp.zeros((N_PAD, PAY), jnp.float32)

    deg_p = _deg_call(mesh=_mesh())(dst, zeros1)
    h1 = _mm1_call(x_pad, W1)
    deg_r = (deg_p.reshape(NC, DEG_PAD)[:, :N_PAD]
             .reshape(NC, _GRID, _RB).transpose(1, 0, 2))
    hs1, dis, inv = _prep_call(deg_r, h1)
    agg1 = _agg_call(mesh=_mesh())(hs1, src, dst, zeros2)
    h2, hs2 = _mid_call(agg1, h1, dis, inv, b1.reshape(1, HID), W2)
    agg2 = _agg_call(mesh=_mesh())(hs2, src, dst, zeros2)
    out = _fin_call(agg2, h2, dis, inv, b2.reshape(1, HID))
    return out[:N_NODES]


# speedup vs baseline: 3.6215x; 3.6215x over previous
import functools

import jax
import jax.numpy as jnp
from jax import lax
from jax.experimental import pallas as pl
from jax.experimental.pallas import tpu as pltpu
from jax.experimental.pallas import tpu_sc as plsc

N_NODES = 10000
N_EDGES = 320000
IN_DIM = 128
HID = 64
PAY = 128

NC, NS = 2, 16
NW = NC * NS
EPT = N_EDGES // NW
K = 80
NBLK = EPT // K
EPT_PAD = 10112
N_PAD = 10112
ROWS_PAD = N_PAD // NS
DEG_PAD = 12288
DEG_ROWS = DEG_PAD // NS

_mesh = functools.partial(
    plsc.VectorSubcoreMesh, core_axis_name="c", subcore_axis_name="s",
    num_cores=NC, num_subcores=NS)


def _deg_body(dst_hbm, zeros_hbm, out_hbm, dst_v, ones_v, deg_sh, sem):
    cid = lax.axis_index("c")
    sid = lax.axis_index("s")
    wid = cid * NS + sid
    pltpu.sync_copy(dst_hbm.at[wid], dst_v)
    for i in range(8):
        ones_v[pl.ds(i * 16, 16)] = jnp.ones((16,), jnp.float32)
    pltpu.sync_copy(zeros_hbm.at[pl.ds(sid * DEG_ROWS, DEG_ROWS)],
                    deg_sh.at[pl.ds(sid * DEG_ROWS, DEG_ROWS)])
    plsc.subcore_barrier()

    def body(j, carry):
        pltpu.async_copy(ones_v.at[pl.ds(0, K)], deg_sh.at[dst_v.at[j]], sem,
                         add=True)
        return carry

    lax.fori_loop(0, NBLK, body, 0)

    def drain(j, carry):
        pltpu.make_async_copy(ones_v.at[pl.ds(0, K)], deg_sh.at[dst_v.at[0]],
                              sem).wait()
        return carry

    lax.fori_loop(0, NBLK, drain, 0)
    plsc.subcore_barrier()
    pltpu.sync_copy(deg_sh.at[pl.ds(sid * DEG_ROWS, DEG_ROWS)],
                    out_hbm.at[pl.ds(cid * DEG_PAD + sid * DEG_ROWS,
                                     DEG_ROWS)])


_deg_call = functools.partial(
    pl.kernel,
    _deg_body,
    out_type=jax.ShapeDtypeStruct((NC * DEG_PAD,), jnp.float32),
    scratch_types=[
        pltpu.VMEM((NBLK, K), jnp.int32),
        pltpu.VMEM((128,), jnp.float32),
        pltpu.VMEM_SHARED((DEG_PAD,), jnp.float32),
        pltpu.SemaphoreType.DMA,
    ],
)


def _agg_body(hs_hbm, src_hbm, dst_hbm, zeros_hbm, out_hbm,
              src_v, dst_v, buf0, buf1, agg_sh, sem0, sem1):
    cid = lax.axis_index("c")
    sid = lax.axis_index("s")
    wid = cid * NS + sid
    pltpu.sync_copy(src_hbm.at[pl.ds(wid * EPT_PAD, EPT_PAD)], src_v)
    pltpu.sync_copy(dst_hbm.at[wid], dst_v)
    pltpu.sync_copy(zeros_hbm.at[pl.ds(sid * ROWS_PAD, ROWS_PAD)],
                    agg_sh.at[pl.ds(sid * ROWS_PAD, ROWS_PAD)])
    plsc.subcore_barrier()

    def src_at(j):
        return src_v.at[pl.ds(pl.multiple_of(j * K, 16), K)]

    pltpu.async_copy(hs_hbm.at[src_at(0)], buf0, sem0)
    pltpu.async_copy(hs_hbm.at[src_at(1)], buf1, sem1)

    def body(j, carry):
        nxt = jnp.minimum(j + 2, NBLK - 1)

        @pl.when(lax.rem(j, 2) == 0)
        def _():
            pltpu.make_async_copy(hs_hbm.at[src_at(0)], buf0, sem0).wait()
            pltpu.sync_copy(buf0, agg_sh.at[dst_v.at[j]], add=True)
            pltpu.async_copy(hs_hbm.at[src_at(nxt)], buf0, sem0)

        @pl.when(lax.rem(j, 2) == 1)
        def _():
            pltpu.make_async_copy(hs_hbm.at[src_at(0)], buf1, sem1).wait()
            pltpu.sync_copy(buf1, agg_sh.at[dst_v.at[j]], add=True)
            pltpu.async_copy(hs_hbm.at[src_at(nxt)], buf1, sem1)

        return carry

    lax.fori_loop(0, NBLK, body, 0)
    pltpu.make_async_copy(hs_hbm.at[src_at(0)], buf0, sem0).wait()
    pltpu.make_async_copy(hs_hbm.at[src_at(0)], buf1, sem1).wait()
    plsc.subcore_barrier()
    pltpu.sync_copy(agg_sh.at[pl.ds(sid * ROWS_PAD, ROWS_PAD)],
                    out_hbm.at[cid, pl.ds(sid * ROWS_PAD, ROWS_PAD)])


_agg_call = functools.partial(
    pl.kernel,
    _agg_body,
    out_type=jax.ShapeDtypeStruct((NC, N_PAD, PAY), jnp.float32),
    scratch_types=[
        pltpu.VMEM((EPT_PAD,), jnp.int32),
        pltpu.VMEM((NBLK, K), jnp.int32),
        pltpu.VMEM((K, PAY), jnp.float32),
        pltpu.VMEM((K, PAY), jnp.float32),
        pltpu.VMEM_SHARED((N_PAD, PAY), jnp.float32),
        pltpu.SemaphoreType.DMA,
        pltpu.SemaphoreType.DMA,
    ],
)


_RB = 632
_GRID = N_PAD // _RB


def _prep_body(degp_ref, x_ref, w1_ref, h1_ref, hs1_ref, dis_ref, inv_ref):
    deg = degp_ref[0, 0, :] + degp_ref[0, 1, :] + 1.0
    inv = 1.0 / deg
    dis = lax.rsqrt(deg)
    h1 = jnp.dot(x_ref[...], w1_ref[...], preferred_element_type=jnp.float32)
    h1_ref[...] = h1
    hs1 = h1 * dis[:, None]
    hs1_ref[...] = jnp.pad(hs1, ((0, 0), (0, PAY - HID)))
    dis_ref[...] = dis[:, None]
    inv_ref[...] = inv[:, None]


def _prep_call(deg_p, x, W1):
    return pl.pallas_call(
        _prep_body,
        grid=(_GRID,),
        in_specs=[
            pl.BlockSpec((1, NC, _RB), lambda i: (i, 0, 0)),
            pl.BlockSpec((_RB, IN_DIM), lambda i: (i, 0)),
            pl.BlockSpec((IN_DIM, HID), lambda i: (0, 0)),
        ],
        out_specs=[
            pl.BlockSpec((_RB, HID), lambda i: (i, 0)),
            pl.BlockSpec((_RB, PAY), lambda i: (i, 0)),
            pl.BlockSpec((_RB, 1), lambda i: (i, 0)),
            pl.BlockSpec((_RB, 1), lambda i: (i, 0)),
        ],
        out_shape=[
            jax.ShapeDtypeStruct((N_PAD, HID), jnp.float32),
            jax.ShapeDtypeStruct((N_PAD, PAY), jnp.float32),
            jax.ShapeDtypeStruct((N_PAD, 1), jnp.float32),
            jax.ShapeDtypeStruct((N_PAD, 1), jnp.float32),
        ],
    )(deg_p, x, W1)


def _mid_body(aggp_ref, h1_ref, dis_ref, inv_ref, b1_ref, w2_ref,
              h2_ref, hs2_ref):
    agg = aggp_ref[0, :, :HID] + aggp_ref[1, :, :HID]
    z1 = jnp.maximum(
        dis_ref[...] * agg + h1_ref[...] * inv_ref[...] + b1_ref[...], 0.0)
    h2 = jnp.dot(z1, w2_ref[...], preferred_element_type=jnp.float32)
    h2_ref[...] = h2
    hs2_ref[...] = jnp.pad(h2 * dis_ref[...], ((0, 0), (0, PAY - HID)))


def _mid_call(agg_p, h1, dis, inv, b1, W2):
    return pl.pallas_call(
        _mid_body,
        grid=(_GRID,),
        in_specs=[
            pl.BlockSpec((NC, _RB, PAY), lambda i: (0, i, 0)),
            pl.BlockSpec((_RB, HID), lambda i: (i, 0)),
            pl.BlockSpec((_RB, 1), lambda i: (i, 0)),
            pl.BlockSpec((_RB, 1), lambda i: (i, 0)),
            pl.BlockSpec((1, HID), lambda i: (0, 0)),
            pl.BlockSpec((HID, HID), lambda i: (0, 0)),
        ],
        out_specs=[
            pl.BlockSpec((_RB, HID), lambda i: (i, 0)),
            pl.BlockSpec((_RB, PAY), lambda i: (i, 0)),
        ],
        out_shape=[
            jax.ShapeDtypeStruct((N_PAD, HID), jnp.float32),
            jax.ShapeDtypeStruct((N_PAD, PAY), jnp.float32),
        ],
    )(agg_p, h1, dis, inv, b1, W2)


def _fin_body(aggp_ref, h2_ref, dis_ref, inv_ref, b2_ref, out_ref):
    agg = aggp_ref[0, :, :HID] + aggp_ref[1, :, :HID]
    out_ref[...] = (dis_ref[...] * agg + h2_ref[...] * inv_ref[...]
                    + b2_ref[...])


def _fin_call(agg_p, h2, dis, inv, b2):
    return pl.pallas_call(
        _fin_body,
        grid=(_GRID,),
        in_specs=[
            pl.BlockSpec((NC, _RB, PAY), lambda i: (0, i, 0)),
            pl.BlockSpec((_RB, HID), lambda i: (i, 0)),
            pl.BlockSpec((_RB, 1), lambda i: (i, 0)),
            pl.BlockSpec((_RB, 1), lambda i: (i, 0)),
            pl.BlockSpec((1, HID), lambda i: (0, 0)),
        ],
        out_specs=pl.BlockSpec((_RB, HID), lambda i: (i, 0)),
        out_shape=jax.ShapeDtypeStruct((N_PAD, HID), jnp.float32),
    )(agg_p, h2, dis, inv, b2)


def kernel(x, edge_index, W1, b1, W2, b2):
    ei = edge_index.astype(jnp.int32)
    src = jnp.pad(ei[0].reshape(NW, EPT), ((0, 0), (0, EPT_PAD - EPT))
                  ).reshape(NW * EPT_PAD)
    dst = ei[1].reshape(NW, NBLK, K)
    x_pad = jnp.pad(x, ((0, N_PAD - N_NODES), (0, 0)))
    zeros1 = jnp.zeros((DEG_PAD,), jnp.float32)
    zeros2 = jnp.zeros((N_PAD, PAY), jnp.float32)

    deg_p = _deg_call(mesh=_mesh())(dst, zeros1)
    deg_r = (deg_p.reshape(NC, DEG_PAD)[:, :N_PAD]
             .reshape(NC, _GRID, _RB).transpose(1, 0, 2))
    h1, hs1, dis, inv = _prep_call(deg_r, x_pad, W1)
    agg1 = _agg_call(mesh=_mesh())(hs1, src, dst, zeros2)
    h2, hs2 = _mid_call(agg1, h1, dis, inv, b1.reshape(1, HID), W2)
    agg2 = _agg_call(mesh=_mesh())(hs2, src, dst, zeros2)
    out = _fin_call(agg2, h2, dis, inv, b2.reshape(1, HID))
    return out[:N_NODES]
